# baseline (speedup 1.0000x reference)
import functools
import math

import jax
import jax.numpy as jnp
from jax.experimental import pallas as pl
from jax.experimental.pallas import tpu as pltpu

B, T, H = 2, 200, 3072
HEADS, HD = 16, 192
FF = 4 * H
NE, NO, NTR = 8, 1000, 100
M = B * T

_f32 = jnp.float32
_bf16 = jnp.bfloat16


def _ln_body(x_ref, g_ref, b_ref, o_ref):
    x = x_ref[...]
    m = jnp.mean(x, axis=-1, keepdims=True)
    xc = x - m
    v = jnp.mean(xc * xc, axis=-1, keepdims=True)
    y = xc / jnp.sqrt(v + 1e-5) * g_ref[...] + b_ref[...]
    o_ref[...] = y.astype(_bf16)


def _layernorm(x, g, b):
    return pl.pallas_call(
        _ln_body,
        out_shape=jax.ShapeDtypeStruct((M, H), _bf16),
    )(x, g.reshape(1, H), b.reshape(1, H))


def _lin_body(x_ref, w_ref, b_ref, o_ref, *, act):
    y = jnp.dot(x_ref[...], w_ref[...].astype(_bf16),
                preferred_element_type=_f32) + b_ref[0]
    if act:
        y = jax.nn.gelu(y)
    o_ref[...] = y.astype(o_ref.dtype)


def _lin_res_body(x_ref, w_ref, b_ref, r_ref, o_ref, *, act, bias_last):
    y = jnp.dot(x_ref[...], w_ref[...].astype(_bf16),
                preferred_element_type=_f32)
    if bias_last:
        y = (y + r_ref[...]) + b_ref[0]
    else:
        y = y + b_ref[0]
        if act:
            y = jax.nn.gelu(y)
        y = y + r_ref[...]
    o_ref[...] = y.astype(o_ref.dtype)


def _linear(x, w, b, *, act=False, residual=None, tn=512, bias_last=False,
            out_dtype=_f32):
    k, n = w.shape
    nt = n // tn
    b3 = b.reshape(nt, 1, tn)
    in_specs = [
        pl.BlockSpec((M, k), lambda j: (0, 0)),
        pl.BlockSpec((k, tn), lambda j: (0, j)),
        pl.BlockSpec((1, 1, tn), lambda j: (j, 0, 0)),
    ]
    args = [x, w, b3]
    if residual is None:
        body = functools.partial(_lin_body, act=act)
    else:
        body = functools.partial(_lin_res_body, act=act, bias_last=bias_last)
        in_specs.append(pl.BlockSpec((M, tn), lambda j: (0, j)))
        args.append(residual)
    return pl.pallas_call(
        body,
        grid=(nt,),
        in_specs=in_specs,
        out_specs=pl.BlockSpec((M, tn), lambda j: (0, j)),
        out_shape=jax.ShapeDtypeStruct((M, n), out_dtype),
    )(*args)


def _attn_body(q_ref, k_ref, v_ref, o_ref):
    q = q_ref[...]
    k = k_ref[...]
    v = v_ref[...]
    lane = jax.lax.broadcasted_iota(jnp.int32, (T, 2 * HD), 1)
    m_a = lane < HD
    zero = jnp.bfloat16(0)
    qa = jnp.where(m_a, q, zero)
    qb = jnp.where(m_a, zero, q)
    cdims = (((1,), (1,)), ((), ()))
    sa = jax.lax.dot_general(qa, k, cdims, preferred_element_type=_f32)
    sb = jax.lax.dot_general(qb, k, cdims, preferred_element_type=_f32)
    scale = jnp.sqrt(jnp.float32(HD))

    def _softmax(s):
        s = s / scale
        s = s - jnp.max(s, axis=-1, keepdims=True)
        e = jnp.exp(s)
        return (e / jnp.sum(e, axis=-1, keepdims=True)).astype(_bf16)

    aa = _softmax(sa)
    ab = _softmax(sb)
    va = jnp.where(m_a, v, zero)
    vb = jnp.where(m_a, zero, v)
    o = (jnp.dot(aa, va, preferred_element_type=_f32)
         + jnp.dot(ab, vb, preferred_element_type=_f32))
    o_ref[...] = o.astype(_bf16)


def _attention(qkv):
    np_ = HEADS // 2
    return pl.pallas_call(
        _attn_body,
        grid=(B, np_),
        in_specs=[
            pl.BlockSpec((T, 2 * HD), lambda b, j: (b, j)),
            pl.BlockSpec((T, 2 * HD), lambda b, j: (b, j + np_)),
            pl.BlockSpec((T, 2 * HD), lambda b, j: (b, j + 2 * np_)),
        ],
        out_specs=pl.BlockSpec((T, 2 * HD), lambda b, j: (b, j)),
        out_shape=jax.ShapeDtypeStruct((M, H), _bf16),
    )(qkv, qkv, qkv)


def _final_body(x_ref, g_ref, b_ref, rw_ref, rb_ref, xtr_ref, probs_ref):
    x = x_ref[...]
    m = jnp.mean(x, axis=-1, keepdims=True)
    xc = x - m
    v = jnp.mean(xc * xc, axis=-1, keepdims=True)
    y = xc / jnp.sqrt(v + 1e-5) * g_ref[...] + b_ref[...]
    nr = B * NTR
    r = jax.lax.broadcasted_iota(jnp.int32, (nr, M), 0)
    c = jax.lax.broadcasted_iota(jnp.int32, (nr, M), 1)
    avg = jnp.where(c // 2 == r, 0.5, 0.0)
    xtr = jnp.dot(avg, y, precision=jax.lax.Precision.HIGHEST,
                  preferred_element_type=_f32)
    logits = jnp.dot(xtr.astype(_bf16), rw_ref[...].astype(_bf16),
                     preferred_element_type=_f32) + rb_ref[...]
    logits = logits - jnp.max(logits, axis=-1, keepdims=True)
    el = jnp.exp(logits)
    p = el / jnp.sum(el, axis=-1, keepdims=True)
    idx = jax.lax.broadcasted_iota(jnp.int32, (nr, NE), 1)
    m1 = jnp.max(p, axis=-1, keepdims=True)
    j1 = jnp.min(jnp.where(p == m1, idx, NE), axis=-1, keepdims=True)
    oh1 = idx == j1
    p2m = jnp.where(oh1, -1.0, p)
    m2 = jnp.max(p2m, axis=-1, keepdims=True)
    j2 = jnp.min(jnp.where(p2m == m2, idx, NE), axis=-1, keepdims=True)
    oh2 = idx == j2
    denom = m1 + m2 + 1e-8
    probs = jnp.where(oh1, m1 / denom, 0.0) + jnp.where(oh2, m2 / denom, 0.0)
    xtr_ref[...] = xtr.astype(_bf16)
    probs_ref[...] = probs


def _final(x, g, b, rw, rb):
    nr = B * NTR
    return pl.pallas_call(
        _final_body,
        out_shape=(
            jax.ShapeDtypeStruct((nr, H), _bf16),
            jax.ShapeDtypeStruct((nr, NE), _f32),
        ),
    )(x, g.reshape(1, H), b.reshape(1, H), rw, rb.reshape(1, NE))


def _expert_body(xtr_ref, w_ref, b_ref, p_ref, o_ref):
    e = pl.program_id(0)
    w = w_ref[0].astype(_bf16)
    y = jnp.dot(xtr_ref[...], w, preferred_element_type=_f32) + b_ref[0]
    nr = B * NTR
    lane = jax.lax.broadcasted_iota(jnp.int32, (nr, NE), 1)
    pe = jnp.sum(jnp.where(lane == e, p_ref[...], 0.0), axis=1, keepdims=True)
    contrib = y * pe

    @pl.when(e == 0)
    def _():
        o_ref[...] = contrib

    @pl.when(e > 0)
    def _():
        o_ref[...] += contrib


def _experts(xtr, exp_w, exp_b, probs):
    nr = B * NTR
    return pl.pallas_call(
        _expert_body,
        grid=(NE,),
        in_specs=[
            pl.BlockSpec((nr, H), lambda e: (0, 0)),
            pl.BlockSpec((1, H, NO), lambda e: (e, 0, 0)),
            pl.BlockSpec((1, 1, NO), lambda e: (e, 0, 0)),
            pl.BlockSpec((nr, NE), lambda e: (0, 0)),
        ],
        out_specs=pl.BlockSpec((nr, NO), lambda e: (0, 0)),
        out_shape=jax.ShapeDtypeStruct((nr, NO), _f32),
    )(xtr, exp_w, exp_b.reshape(NE, 1, NO), probs)


def kernel(feat, params):
    p = params
    x0 = feat.reshape(B, H, T).transpose(0, 2, 1).reshape(M, H)
    xn = _layernorm(x0, p['proj_ln_g'], p['proj_ln_b'])
    pos = jnp.broadcast_to(p['pos'][0, :T], (B, T, H)).reshape(M, H)
    x = _linear(xn, p['proj_w'], p['proj_b'], act=True, residual=pos)
    for lp in p['layers']:
        xn = _layernorm(x, lp['ln1_g'], lp['ln1_b'])
        qkv = _linear(xn, lp['wqkv'], lp['bqkv'], out_dtype=_bf16)
        ao = _attention(qkv)
        x = _linear(ao, lp['wo'], lp['bo'], residual=x)
        xn = _layernorm(x, lp['ln2_g'], lp['ln2_b'])
        h = _linear(xn, lp['w1'], lp['b1'], act=True, out_dtype=_bf16)
        x = _linear(h, lp['w2'], lp['b2'], residual=x, tn=256, bias_last=True)
    xtr, probs = _final(x, p['fin_g'], p['fin_b'], p['router_w'], p['router_b'])
    y = _experts(xtr, p['exp_w'], p['exp_b'], probs)
    return y.reshape(B, NTR, NO).transpose(0, 2, 1)


# speedup vs baseline: 1.0103x; 1.0103x over previous
import functools
import math

import jax
import jax.numpy as jnp
from jax.experimental import pallas as pl
from jax.experimental.pallas import tpu as pltpu

B, T, H = 2, 200, 3072
HEADS, HD = 16, 192
FF = 4 * H
NE, NO, NTR = 8, 1000, 100
M = B * T

_f32 = jnp.float32
_bf16 = jnp.bfloat16


def _ln_body(x_ref, g_ref, b_ref, o_ref):
    x = x_ref[...]
    m = jnp.mean(x, axis=-1, keepdims=True)
    xc = x - m
    v = jnp.mean(xc * xc, axis=-1, keepdims=True)
    y = xc / jnp.sqrt(v + 1e-5) * g_ref[...] + b_ref[...]
    o_ref[...] = y.astype(_bf16)


def _layernorm(x, g, b):
    return pl.pallas_call(
        _ln_body,
        out_shape=jax.ShapeDtypeStruct((M, H), _bf16),
    )(x, g.reshape(1, H), b.reshape(1, H))


def _lin_body(x_ref, w_ref, b_ref, o_ref, *, act):
    y = jnp.dot(x_ref[...], w_ref[...].astype(_bf16),
                preferred_element_type=_f32) + b_ref[0]
    if act:
        y = jax.nn.gelu(y)
    o_ref[...] = y.astype(o_ref.dtype)


def _lin_res_body(x_ref, w_ref, b_ref, r_ref, o_ref, *, act, bias_last):
    y = jnp.dot(x_ref[...], w_ref[...].astype(_bf16),
                preferred_element_type=_f32)
    if bias_last:
        y = (y + r_ref[...]) + b_ref[0]
    else:
        y = y + b_ref[0]
        if act:
            y = jax.nn.gelu(y)
        y = y + r_ref[...]
    o_ref[...] = y.astype(o_ref.dtype)


def _linear(x, w, b, *, act=False, residual=None, tn=1024, bias_last=False,
            out_dtype=_f32):
    k, n = w.shape
    nt = n // tn
    b3 = b.reshape(nt, 1, tn)
    in_specs = [
        pl.BlockSpec((M, k), lambda j: (0, 0)),
        pl.BlockSpec((k, tn), lambda j: (0, j)),
        pl.BlockSpec((1, 1, tn), lambda j: (j, 0, 0)),
    ]
    args = [x, w, b3]
    if residual is None:
        body = functools.partial(_lin_body, act=act)
    else:
        body = functools.partial(_lin_res_body, act=act, bias_last=bias_last)
        in_specs.append(pl.BlockSpec((M, tn), lambda j: (0, j)))
        args.append(residual)
    return pl.pallas_call(
        body,
        grid=(nt,),
        in_specs=in_specs,
        out_specs=pl.BlockSpec((M, tn), lambda j: (0, j)),
        out_shape=jax.ShapeDtypeStruct((M, n), out_dtype),
    )(*args)


def _attn_body(q_ref, k_ref, v_ref, o_ref):
    q = q_ref[...]
    k = k_ref[...]
    v = v_ref[...]
    lane = jax.lax.broadcasted_iota(jnp.int32, (T, 2 * HD), 1)
    m_a = lane < HD
    zero = jnp.bfloat16(0)
    qa = jnp.where(m_a, q, zero)
    qb = jnp.where(m_a, zero, q)
    cdims = (((1,), (1,)), ((), ()))
    sa = jax.lax.dot_general(qa, k, cdims, preferred_element_type=_f32)
    sb = jax.lax.dot_general(qb, k, cdims, preferred_element_type=_f32)
    scale = jnp.sqrt(jnp.float32(HD))

    def _softmax(s):
        s = s / scale
        s = s - jnp.max(s, axis=-1, keepdims=True)
        e = jnp.exp(s)
        return (e / jnp.sum(e, axis=-1, keepdims=True)).astype(_bf16)

    aa = _softmax(sa)
    ab = _softmax(sb)
    va = jnp.where(m_a, v, zero)
    vb = jnp.where(m_a, zero, v)
    o = (jnp.dot(aa, va, preferred_element_type=_f32)
         + jnp.dot(ab, vb, preferred_element_type=_f32))
    o_ref[...] = o.astype(_bf16)


def _attention(qkv):
    np_ = HEADS // 2
    return pl.pallas_call(
        _attn_body,
        grid=(B, np_),
        in_specs=[
            pl.BlockSpec((T, 2 * HD), lambda b, j: (b, j)),
            pl.BlockSpec((T, 2 * HD), lambda b, j: (b, j + np_)),
            pl.BlockSpec((T, 2 * HD), lambda b, j: (b, j + 2 * np_)),
        ],
        out_specs=pl.BlockSpec((T, 2 * HD), lambda b, j: (b, j)),
        out_shape=jax.ShapeDtypeStruct((M, H), _bf16),
    )(qkv, qkv, qkv)


def _final_body(x_ref, g_ref, b_ref, rw_ref, rb_ref, xtr_ref, probs_ref):
    x = x_ref[...]
    m = jnp.mean(x, axis=-1, keepdims=True)
    xc = x - m
    v = jnp.mean(xc * xc, axis=-1, keepdims=True)
    y = xc / jnp.sqrt(v + 1e-5) * g_ref[...] + b_ref[...]
    nr = B * NTR
    r = jax.lax.broadcasted_iota(jnp.int32, (nr, M), 0)
    c = jax.lax.broadcasted_iota(jnp.int32, (nr, M), 1)
    avg = jnp.where(c // 2 == r, 0.5, 0.0)
    xtr = jnp.dot(avg, y, precision=jax.lax.Precision.HIGHEST,
                  preferred_element_type=_f32)
    logits = jnp.dot(xtr.astype(_bf16), rw_ref[...].astype(_bf16),
                     preferred_element_type=_f32) + rb_ref[...]
    logits = logits - jnp.max(logits, axis=-1, keepdims=True)
    el = jnp.exp(logits)
    p = el / jnp.sum(el, axis=-1, keepdims=True)
    idx = jax.lax.broadcasted_iota(jnp.int32, (nr, NE), 1)
    m1 = jnp.max(p, axis=-1, keepdims=True)
    j1 = jnp.min(jnp.where(p == m1, idx, NE), axis=-1, keepdims=True)
    oh1 = idx == j1
    p2m = jnp.where(oh1, -1.0, p)
    m2 = jnp.max(p2m, axis=-1, keepdims=True)
    j2 = jnp.min(jnp.where(p2m == m2, idx, NE), axis=-1, keepdims=True)
    oh2 = idx == j2
    denom = m1 + m2 + 1e-8
    probs = jnp.where(oh1, m1 / denom, 0.0) + jnp.where(oh2, m2 / denom, 0.0)
    xtr_ref[...] = xtr.astype(_bf16)
    probs_ref[...] = probs


def _final(x, g, b, rw, rb):
    nr = B * NTR
    return pl.pallas_call(
        _final_body,
        out_shape=(
            jax.ShapeDtypeStruct((nr, H), _bf16),
            jax.ShapeDtypeStruct((nr, NE), _f32),
        ),
    )(x, g.reshape(1, H), b.reshape(1, H), rw, rb.reshape(1, NE))


def _expert_body(xtr_ref, w_ref, b_ref, p_ref, o_ref):
    e = pl.program_id(0)
    w = w_ref[0].astype(_bf16)
    y = jnp.dot(xtr_ref[...], w, preferred_element_type=_f32) + b_ref[0]
    nr = B * NTR
    lane = jax.lax.broadcasted_iota(jnp.int32, (nr, NE), 1)
    pe = jnp.sum(jnp.where(lane == e, p_ref[...], 0.0), axis=1, keepdims=True)
    contrib = y * pe

    @pl.when(e == 0)
    def _():
        o_ref[...] = contrib

    @pl.when(e > 0)
    def _():
        o_ref[...] += contrib


def _experts(xtr, exp_w, exp_b, probs):
    nr = B * NTR
    return pl.pallas_call(
        _expert_body,
        grid=(NE,),
        in_specs=[
            pl.BlockSpec((nr, H), lambda e: (0, 0)),
            pl.BlockSpec((1, H, NO), lambda e: (e, 0, 0)),
            pl.BlockSpec((1, 1, NO), lambda e: (e, 0, 0)),
            pl.BlockSpec((nr, NE), lambda e: (0, 0)),
        ],
        out_specs=pl.BlockSpec((nr, NO), lambda e: (0, 0)),
        out_shape=jax.ShapeDtypeStruct((nr, NO), _f32),
    )(xtr, exp_w, exp_b.reshape(NE, 1, NO), probs)


def kernel(feat, params):
    p = params
    x0 = feat.reshape(B, H, T).transpose(0, 2, 1).reshape(M, H)
    xn = _layernorm(x0, p['proj_ln_g'], p['proj_ln_b'])
    pos = jnp.broadcast_to(p['pos'][0, :T], (B, T, H)).reshape(M, H)
    x = _linear(xn, p['proj_w'], p['proj_b'], act=True, residual=pos)
    for lp in p['layers']:
        xn = _layernorm(x, lp['ln1_g'], lp['ln1_b'])
        qkv = _linear(xn, lp['wqkv'], lp['bqkv'], out_dtype=_bf16)
        ao = _attention(qkv)
        x = _linear(ao, lp['wo'], lp['bo'], residual=x)
        xn = _layernorm(x, lp['ln2_g'], lp['ln2_b'])
        h = _linear(xn, lp['w1'], lp['b1'], act=True, out_dtype=_bf16)
        x = _linear(h, lp['w2'], lp['b2'], residual=x, tn=256, bias_last=True)
    xtr, probs = _final(x, p['fin_g'], p['fin_b'], p['router_w'], p['router_b'])
    y = _experts(xtr, p['exp_w'], p['exp_b'], probs)
    return y.reshape(B, NTR, NO).transpose(0, 2, 1)
